# Initial kernel scaffold; baseline (speedup 1.0000x reference)
import jax
import jax.numpy as jnp
from jax.experimental import pallas as pl


def kernel(goals, hyps, segment_ids, W1, b1, W2, b2, W3, b3):
    raise NotImplementedError("write your pallas kernel here")


# speedup vs baseline: 3.3178x; 3.3178x over previous
import functools

import jax
import jax.numpy as jnp
from jax import lax
from jax.experimental import pallas as pl
from jax.experimental.pallas import tpu as pltpu
from jax.experimental.pallas import tpu_sc as plsc

B, D, E = 10000, 128, 320000
H = 256
BP = 10240

NC, NS = 2, 16
NW = NC * NS
ROWS_PER_TILE = E // NW
WIN = 80
NWIN = ROWS_PER_TILE // WIN
ZR = 128
ROWS_PER_TILE_B = BP // NS
NZ = ROWS_PER_TILE_B // ZR


def _segment_sum_sc(hyps, seg_ids):
  mesh = plsc.VectorSubcoreMesh(
      core_axis_name="c", subcore_axis_name="s", num_cores=NC,
      num_subcores=NS)

  @functools.partial(
      pl.kernel,
      out_type=jax.ShapeDtypeStruct((NC, BP, D), jnp.float32),
      mesh=mesh,
      scratch_types=[
          pltpu.VMEM((WIN, D), jnp.float32),
          pltpu.VMEM((WIN,), jnp.int32),
          pltpu.VMEM((ZR, D), jnp.float32),
          pltpu.VMEM_SHARED((BP, D), jnp.float32),
      ],
  )
  def k(hyps_hbm, ids_hbm, out_hbm, rows_v, idx_v, zbuf_v, acc_sh):
    cid = lax.axis_index("c")
    sid = lax.axis_index("s")

    def zero_body(t, _):
      i = t // (D // 16)
      j = t % (D // 16)
      zbuf_v[i, pl.ds(j * 16, 16)] = jnp.zeros((16,), jnp.float32)
      return 0
    lax.fori_loop(0, ZR * (D // 16), zero_body, 0)

    tile_out_base = sid * ROWS_PER_TILE_B
    for z in range(NZ):
      pltpu.sync_copy(zbuf_v, acc_sh.at[pl.ds(tile_out_base + z * ZR, ZR)])
    plsc.subcore_barrier()

    chunk_base = (cid * NS + sid) * ROWS_PER_TILE

    def win_body(w, _):
      off = chunk_base + w * WIN
      pltpu.sync_copy(hyps_hbm.at[pl.ds(off, WIN)], rows_v)
      pltpu.sync_copy(ids_hbm.at[pl.ds(off, WIN)], idx_v)
      pltpu.sync_copy(rows_v, acc_sh.at[idx_v], add=True)
      return 0
    lax.fori_loop(0, NWIN, win_body, 0)

    plsc.subcore_barrier()

    for z in range(NZ):
      base = tile_out_base + z * ZR
      pltpu.sync_copy(acc_sh.at[pl.ds(base, ZR)], zbuf_v)
      pltpu.sync_copy(zbuf_v, out_hbm.at[cid, pl.ds(base, ZR)])

  return k(hyps, seg_ids)


def _mlp_body(goals_ref, p0_ref, p1_ref, w1_ref, b1_ref, w2_ref, b2_ref,
              w3_ref, b3_ref, out_ref):
  hsum = p0_ref[...] + p1_ref[...]
  x = jnp.concatenate([goals_ref[...], hsum], axis=1)
  z1 = lax.dot_general(x, w1_ref[...], (((1,), (1,)), ((), ())),
                       precision=lax.Precision.HIGHEST,
                       preferred_element_type=jnp.float32) + b1_ref[...]
  a1 = jnp.where(z1 >= 0, z1, 0.01 * z1)
  z2 = lax.dot_general(a1, w2_ref[...], (((1,), (1,)), ((), ())),
                       precision=lax.Precision.HIGHEST,
                       preferred_element_type=jnp.float32) + b2_ref[...]
  a2 = jnp.where(z2 >= 0, z2, 0.01 * z2)
  z3 = jnp.sum(a2 * w3_ref[...], axis=1, keepdims=True) + b3_ref[0, 0]
  out_ref[...] = jax.nn.sigmoid(z3)


def _mlp_tc(goals, p0, p1, W1, b1, W2, b2, W3, b3):
  R = 1000
  grid = B // R
  full = lambda shape: pl.BlockSpec(shape, lambda i: (0, 0))
  return pl.pallas_call(
      _mlp_body,
      grid=(grid,),
      in_specs=[
          pl.BlockSpec((R, D), lambda i: (i, 0)),
          pl.BlockSpec((R, D), lambda i: (i, 0)),
          pl.BlockSpec((R, D), lambda i: (i, 0)),
          full(W1.shape),
          full((1, H)),
          full(W2.shape),
          full((1, H)),
          full(W3.shape),
          full((1, 1)),
      ],
      out_specs=pl.BlockSpec((R, 1), lambda i: (i, 0)),
      out_shape=jax.ShapeDtypeStruct((B, 1), jnp.float32),
  )(goals, p0, p1, W1, b1.reshape(1, H), W2, b2.reshape(1, H), W3,
    b3.reshape(1, 1))


def kernel(goals, hyps, segment_ids, W1, b1, W2, b2, W3, b3):
  seg = segment_ids.astype(jnp.int32)
  partials = _segment_sum_sc(hyps, seg)
  return _mlp_tc(goals, partials[0], partials[1], W1, b1, W2, b2, W3, b3)


# speedup vs baseline: 5.7720x; 1.7397x over previous
import functools

import jax
import jax.numpy as jnp
from jax import lax
from jax.experimental import pallas as pl
from jax.experimental.pallas import tpu as pltpu
from jax.experimental.pallas import tpu_sc as plsc

B, D, E = 10000, 128, 320000
H = 256
BP = 10240

NC, NS = 2, 16
NW = NC * NS
ROWS_PER_TILE = E // NW
WIN = 80
NWIN = ROWS_PER_TILE // WIN
ZR = 64
ROWS_PER_TILE_B = BP // NS
NZ = ROWS_PER_TILE_B // ZR


def _segment_sum_sc(hyps, seg_ids):
  mesh = plsc.VectorSubcoreMesh(
      core_axis_name="c", subcore_axis_name="s", num_cores=NC,
      num_subcores=NS)

  @functools.partial(
      pl.kernel,
      out_type=jax.ShapeDtypeStruct((NC, BP, D), jnp.float32),
      mesh=mesh,
      scratch_types=[
          pltpu.VMEM((2, WIN, D), jnp.float32),
          pltpu.VMEM((NWIN, WIN), jnp.int32),
          pltpu.VMEM((ZR, D), jnp.float32),
          pltpu.VMEM_SHARED((BP, D), jnp.float32),
          pltpu.SemaphoreType.DMA,
          pltpu.SemaphoreType.DMA,
          pltpu.SemaphoreType.DMA,
      ],
  )
  def k(hyps_hbm, ids_hbm, out_hbm, rows_v, idx_v, zbuf_v, acc_sh,
        sem0, sem1, sem_idx):
    cid = lax.axis_index("c")
    sid = lax.axis_index("s")
    tile = cid * NS + sid
    sems = (sem0, sem1)

    idx_load = pltpu.async_copy(ids_hbm.at[tile], idx_v, sem_idx)

    def zero_body(t, _):
      i = t // (D // 16)
      j = t % (D // 16)
      zbuf_v[i, pl.ds(j * 16, 16)] = jnp.zeros((16,), jnp.float32)
      return 0
    lax.fori_loop(0, ZR * (D // 16), zero_body, 0)

    tile_out_base = sid * ROWS_PER_TILE_B
    for z in range(NZ):
      pltpu.sync_copy(zbuf_v, acc_sh.at[pl.ds(tile_out_base + z * ZR, ZR)])

    chunk_base = tile * ROWS_PER_TILE

    def start_load(w, b):
      pltpu.async_copy(
          hyps_hbm.at[pl.ds(chunk_base + w * WIN, WIN)], rows_v.at[b],
          sems[b])

    def wait_load(w, b):
      pltpu.make_async_copy(
          hyps_hbm.at[pl.ds(chunk_base + w * WIN, WIN)], rows_v.at[b],
          sems[b]).wait()

    def scatter(w, b):
      pltpu.sync_copy(rows_v.at[b], acc_sh.at[idx_v.at[w]], add=True)

    start_load(0, 0)
    idx_load.wait()
    plsc.subcore_barrier()

    def win_body(i, _):
      w = i * 2
      start_load(w + 1, 1)
      wait_load(w, 0)
      scatter(w, 0)
      start_load(w + 2, 0)
      wait_load(w + 1, 1)
      scatter(w + 1, 1)
      return 0
    lax.fori_loop(0, (NWIN - 1) // 2, win_body, 0)
    wait_load(NWIN - 1, 0)
    scatter(NWIN - 1, 0)

    plsc.subcore_barrier()

    for z in range(NZ):
      base = tile_out_base + z * ZR
      pltpu.sync_copy(acc_sh.at[pl.ds(base, ZR)], zbuf_v)
      pltpu.sync_copy(zbuf_v, out_hbm.at[cid, pl.ds(base, ZR)])

  return k(hyps, seg_ids)


def _mlp_body(goals_ref, p0_ref, p1_ref, w1_ref, b1_ref, w2_ref, b2_ref,
              w3_ref, b3_ref, out_ref):
  hsum = p0_ref[...] + p1_ref[...]
  x = jnp.concatenate([goals_ref[...], hsum], axis=1)
  z1 = lax.dot_general(x, w1_ref[...], (((1,), (1,)), ((), ())),
                       precision=lax.Precision.HIGHEST,
                       preferred_element_type=jnp.float32) + b1_ref[...]
  a1 = jnp.where(z1 >= 0, z1, 0.01 * z1)
  z2 = lax.dot_general(a1, w2_ref[...], (((1,), (1,)), ((), ())),
                       precision=lax.Precision.HIGHEST,
                       preferred_element_type=jnp.float32) + b2_ref[...]
  a2 = jnp.where(z2 >= 0, z2, 0.01 * z2)
  z3 = jnp.sum(a2 * w3_ref[...], axis=1, keepdims=True) + b3_ref[0, 0]
  out_ref[...] = jax.nn.sigmoid(z3)


def _mlp_tc(goals, p0, p1, W1, b1, W2, b2, W3, b3):
  R = 1000
  grid = B // R
  full = lambda shape: pl.BlockSpec(shape, lambda i: (0, 0))
  return pl.pallas_call(
      _mlp_body,
      grid=(grid,),
      in_specs=[
          pl.BlockSpec((R, D), lambda i: (i, 0)),
          pl.BlockSpec((R, D), lambda i: (i, 0)),
          pl.BlockSpec((R, D), lambda i: (i, 0)),
          full(W1.shape),
          full((1, H)),
          full(W2.shape),
          full((1, H)),
          full(W3.shape),
          full((1, 1)),
      ],
      out_specs=pl.BlockSpec((R, 1), lambda i: (i, 0)),
      out_shape=jax.ShapeDtypeStruct((B, 1), jnp.float32),
  )(goals, p0, p1, W1, b1.reshape(1, H), W2, b2.reshape(1, H), W3,
    b3.reshape(1, 1))


def kernel(goals, hyps, segment_ids, W1, b1, W2, b2, W3, b3):
  seg = segment_ids.astype(jnp.int32).reshape(NW, NWIN, WIN)
  partials = _segment_sum_sc(hyps, seg)
  return _mlp_tc(goals, partials[0], partials[1], W1, b1, W2, b2, W3, b3)


# speedup vs baseline: 6.9158x; 1.1982x over previous
import functools

import jax
import jax.numpy as jnp
from jax import lax
from jax.experimental import pallas as pl
from jax.experimental.pallas import tpu as pltpu
from jax.experimental.pallas import tpu_sc as plsc

B, D, E = 10000, 128, 320000
H = 256
BP = 10240

NC, NS = 2, 16
NW = NC * NS
ROWS_PER_TILE = E // NW
WIN = 80
NWIN = ROWS_PER_TILE // WIN
ZR = 64
ROWS_PER_TILE_B = BP // NS
NZ = ROWS_PER_TILE_B // ZR


def _segment_sum_sc(hyps, seg_ids):
  mesh = plsc.VectorSubcoreMesh(
      core_axis_name="c", subcore_axis_name="s", num_cores=NC,
      num_subcores=NS)

  @functools.partial(
      pl.kernel,
      out_type=jax.ShapeDtypeStruct((NC, BP, D), jnp.float32),
      mesh=mesh,
      scratch_types=[
          pltpu.VMEM((2, WIN, D), jnp.float32),
          pltpu.VMEM((NWIN, WIN), jnp.int32),
          pltpu.VMEM((ZR, D), jnp.float32),
          pltpu.VMEM_SHARED((BP, D), jnp.float32),
          pltpu.SemaphoreType.DMA,
          pltpu.SemaphoreType.DMA,
          pltpu.SemaphoreType.DMA,
      ],
  )
  def k(hyps_hbm, ids_hbm, out_hbm, rows_v, idx_v, zbuf_v, acc_sh,
        sem0, sem1, sem_idx):
    cid = lax.axis_index("c")
    sid = lax.axis_index("s")
    tile = cid * NS + sid
    sems = (sem0, sem1)

    idx_load = pltpu.async_copy(ids_hbm.at[tile], idx_v, sem_idx)

    def zero_body(t, _):
      i = t // (D // 16)
      j = t % (D // 16)
      zbuf_v[i, pl.ds(j * 16, 16)] = jnp.zeros((16,), jnp.float32)
      return 0
    lax.fori_loop(0, ZR * (D // 16), zero_body, 0)

    tile_out_base = sid * ROWS_PER_TILE_B
    for z in range(NZ):
      pltpu.sync_copy(zbuf_v, acc_sh.at[pl.ds(tile_out_base + z * ZR, ZR)])

    chunk_base = tile * ROWS_PER_TILE

    def start_load(w, b):
      pltpu.async_copy(
          hyps_hbm.at[pl.ds(chunk_base + w * WIN, WIN)], rows_v.at[b],
          sems[b])

    def wait_load(w, b):
      pltpu.make_async_copy(
          hyps_hbm.at[pl.ds(chunk_base + w * WIN, WIN)], rows_v.at[b],
          sems[b]).wait()

    def scatter(w, b):
      pltpu.sync_copy(rows_v.at[b], acc_sh.at[idx_v.at[w]], add=True)

    start_load(0, 0)
    idx_load.wait()
    plsc.subcore_barrier()

    def win_body(i, _):
      w = i * 2
      start_load(w + 1, 1)
      wait_load(w, 0)
      scatter(w, 0)
      start_load(w + 2, 0)
      wait_load(w + 1, 1)
      scatter(w + 1, 1)
      return 0
    lax.fori_loop(0, (NWIN - 1) // 2, win_body, 0)
    wait_load(NWIN - 1, 0)
    scatter(NWIN - 1, 0)

    plsc.subcore_barrier()

    for z in range(NZ):
      base = tile_out_base + z * ZR
      pltpu.sync_copy(acc_sh.at[pl.ds(base, ZR)], zbuf_v)
      pltpu.sync_copy(zbuf_v, out_hbm.at[cid, pl.ds(base, ZR)])

  return k(hyps, seg_ids)


def _mlp_body(goals_ref, p_ref, w1_ref, b1_ref, w2_ref, b2_ref,
              w3_ref, b3_ref, out_ref):
  hsum = p_ref[0] + p_ref[1]
  x = jnp.concatenate([goals_ref[...], hsum], axis=1)
  z1 = lax.dot_general(x, w1_ref[...], (((1,), (1,)), ((), ())),
                       precision=lax.Precision.DEFAULT,
                       preferred_element_type=jnp.float32) + b1_ref[...]
  a1 = jnp.where(z1 >= 0, z1, 0.01 * z1)
  z2 = lax.dot_general(a1, w2_ref[...], (((1,), (1,)), ((), ())),
                       precision=lax.Precision.DEFAULT,
                       preferred_element_type=jnp.float32) + b2_ref[...]
  a2 = jnp.where(z2 >= 0, z2, 0.01 * z2)
  z3 = jnp.sum(a2 * w3_ref[...], axis=1, keepdims=True) + b3_ref[0, 0]
  out_ref[...] = jax.nn.sigmoid(z3)


def _mlp_tc(goals, partials, W1, b1, W2, b2, W3, b3):
  R = 1000
  grid = B // R
  full = lambda shape: pl.BlockSpec(shape, lambda i: (0, 0))
  return pl.pallas_call(
      _mlp_body,
      grid=(grid,),
      in_specs=[
          pl.BlockSpec((R, D), lambda i: (i, 0)),
          pl.BlockSpec((NC, R, D), lambda i: (0, i, 0)),
          full(W1.shape),
          full((1, H)),
          full(W2.shape),
          full((1, H)),
          full(W3.shape),
          full((1, 1)),
      ],
      out_specs=pl.BlockSpec((R, 1), lambda i: (i, 0)),
      out_shape=jax.ShapeDtypeStruct((B, 1), jnp.float32),
  )(goals, partials, W1, b1.reshape(1, H), W2, b2.reshape(1, H), W3,
    b3.reshape(1, 1))


def kernel(goals, hyps, segment_ids, W1, b1, W2, b2, W3, b3):
  seg = segment_ids.astype(jnp.int32).reshape(NW, NWIN, WIN)
  partials = _segment_sum_sc(hyps, seg)
  return _mlp_tc(goals, partials, W1, b1, W2, b2, W3, b3)


# speedup vs baseline: 7.1154x; 1.0289x over previous
import functools

import jax
import jax.numpy as jnp
from jax import lax
from jax.experimental import pallas as pl
from jax.experimental.pallas import tpu as pltpu
from jax.experimental.pallas import tpu_sc as plsc

B, D, E = 10000, 128, 320000
H = 256
BP = 10240

NC, NS = 2, 16
NW = NC * NS
ROWS_PER_TILE = E // NW
WIN = 80
NWIN = ROWS_PER_TILE // WIN
ZR = WIN
ROWS_PER_TILE_B = BP // NS
NZ = ROWS_PER_TILE_B // ZR


def _segment_sum_sc(hyps, seg_ids):
  mesh = plsc.VectorSubcoreMesh(
      core_axis_name="c", subcore_axis_name="s", num_cores=NC,
      num_subcores=NS)

  @functools.partial(
      pl.kernel,
      out_type=jax.ShapeDtypeStruct((NC, BP, D), jnp.float32),
      mesh=mesh,
      scratch_types=[
          pltpu.VMEM((3, WIN, D), jnp.float32),
          pltpu.VMEM((NWIN, WIN), jnp.int32),
          pltpu.VMEM_SHARED((BP, D), jnp.float32),
          pltpu.SemaphoreType.DMA,
          pltpu.SemaphoreType.DMA,
          pltpu.SemaphoreType.DMA,
          pltpu.SemaphoreType.DMA,
          pltpu.SemaphoreType.DMA,
          pltpu.SemaphoreType.DMA,
          pltpu.SemaphoreType.DMA,
      ],
  )
  def k(hyps_hbm, ids_hbm, out_hbm, rows_v, idx_v, acc_sh,
        seml0, seml1, seml2, sems0, sems1, sems2, sem_idx):
    cid = lax.axis_index("c")
    sid = lax.axis_index("s")
    tile = cid * NS + sid
    load_sems = (seml0, seml1, seml2)
    scat_sems = (sems0, sems1, sems2)
    chunk_base = tile * ROWS_PER_TILE

    def start_load(w, b):
      pltpu.async_copy(
          hyps_hbm.at[pl.ds(chunk_base + w * WIN, WIN)], rows_v.at[b],
          load_sems[b])

    def wait_load(w, b):
      pltpu.make_async_copy(
          hyps_hbm.at[pl.ds(chunk_base + w * WIN, WIN)], rows_v.at[b],
          load_sems[b]).wait()

    def start_scatter(w, b):
      pltpu.async_copy(rows_v.at[b], acc_sh.at[idx_v.at[w]], scat_sems[b],
                       add=True)

    def wait_scatter(w, b):
      pltpu.make_async_copy(rows_v.at[b], acc_sh.at[idx_v.at[w]],
                            scat_sems[b]).wait()

    idx_load = pltpu.async_copy(ids_hbm.at[tile], idx_v, sem_idx)
    start_load(0, 0)
    start_load(1, 1)

    def zero_body(t, _):
      i = t // (D // 16)
      j = t % (D // 16)
      rows_v[2, i, pl.ds(j * 16, 16)] = jnp.zeros((16,), jnp.float32)
      return 0
    lax.fori_loop(0, ZR * (D // 16), zero_body, 0)

    tile_out_base = sid * ROWS_PER_TILE_B
    for z in range(NZ):
      pltpu.sync_copy(rows_v.at[2],
                      acc_sh.at[pl.ds(tile_out_base + z * ZR, ZR)])

    idx_load.wait()
    plsc.subcore_barrier()

    def win_body(i, _):
      for j in range(3):
        w = i * 3 + j
        b = j
        wait_load(w, b)
        start_scatter(w, b)

        @pl.when(w >= 1)
        def _():
          wait_scatter(w - 1, (j - 1) % 3)
        start_load(w + 2, (j + 2) % 3)
      return 0
    lax.fori_loop(0, (NWIN - 2) // 3, win_body, 0)

    w0, w1 = NWIN - 2, NWIN - 1
    b0, b1 = w0 % 3, w1 % 3
    wait_load(w0, b0)
    start_scatter(w0, b0)
    wait_load(w1, b1)
    start_scatter(w1, b1)
    wait_scatter(w1 - 2, (b1 - 2) % 3)
    wait_scatter(w0, b0)
    wait_scatter(w1, b1)

    plsc.subcore_barrier()

    for z in range(NZ):
      base = tile_out_base + z * ZR
      pltpu.sync_copy(acc_sh.at[pl.ds(base, ZR)], rows_v.at[2])
      pltpu.sync_copy(rows_v.at[2], out_hbm.at[cid, pl.ds(base, ZR)])

  return k(hyps, seg_ids)


def _mlp_body(goals_ref, p_ref, w1_ref, b1_ref, w2_ref, b2_ref,
              w3_ref, b3_ref, out_ref):
  hsum = p_ref[0] + p_ref[1]
  x = jnp.concatenate([goals_ref[...], hsum], axis=1)
  z1 = lax.dot_general(x, w1_ref[...], (((1,), (1,)), ((), ())),
                       precision=lax.Precision.DEFAULT,
                       preferred_element_type=jnp.float32) + b1_ref[...]
  a1 = jnp.where(z1 >= 0, z1, 0.01 * z1)
  z2 = lax.dot_general(a1, w2_ref[...], (((1,), (1,)), ((), ())),
                       precision=lax.Precision.DEFAULT,
                       preferred_element_type=jnp.float32) + b2_ref[...]
  a2 = jnp.where(z2 >= 0, z2, 0.01 * z2)
  z3 = jnp.sum(a2 * w3_ref[...], axis=1, keepdims=True) + b3_ref[0, 0]
  out_ref[...] = jax.nn.sigmoid(z3)


def _mlp_tc(goals, partials, W1, b1, W2, b2, W3, b3):
  R = 1000
  grid = B // R
  full = lambda shape: pl.BlockSpec(shape, lambda i: (0, 0))
  return pl.pallas_call(
      _mlp_body,
      grid=(grid,),
      in_specs=[
          pl.BlockSpec((R, D), lambda i: (i, 0)),
          pl.BlockSpec((NC, R, D), lambda i: (0, i, 0)),
          full(W1.shape),
          full((1, H)),
          full(W2.shape),
          full((1, H)),
          full(W3.shape),
          full((1, 1)),
      ],
      out_specs=pl.BlockSpec((R, 1), lambda i: (i, 0)),
      out_shape=jax.ShapeDtypeStruct((B, 1), jnp.float32),
  )(goals, partials, W1, b1.reshape(1, H), W2, b2.reshape(1, H), W3,
    b3.reshape(1, 1))


def kernel(goals, hyps, segment_ids, W1, b1, W2, b2, W3, b3):
  seg = segment_ids.astype(jnp.int32).reshape(NW, NWIN, WIN)
  partials = _segment_sum_sc(hyps, seg)
  return _mlp_tc(goals, partials, W1, b1, W2, b2, W3, b3)


# speedup vs baseline: 7.1819x; 1.0093x over previous
import functools

import jax
import jax.numpy as jnp
from jax import lax
from jax.experimental import pallas as pl
from jax.experimental.pallas import tpu as pltpu
from jax.experimental.pallas import tpu_sc as plsc

B, D, E = 10000, 128, 320000
H = 256
BP = 10240

NC, NS = 2, 16
NW = NC * NS
ROWS_PER_TILE = E // NW
WIN = 80
NWIN = ROWS_PER_TILE // WIN
ZR = WIN
ROWS_PER_TILE_B = BP // NS
NZ = ROWS_PER_TILE_B // ZR


def _segment_sum_sc(hyps, seg_ids):
  mesh = plsc.VectorSubcoreMesh(
      core_axis_name="c", subcore_axis_name="s", num_cores=NC,
      num_subcores=NS)

  @functools.partial(
      pl.kernel,
      out_type=jax.ShapeDtypeStruct((NC, BP, D), jnp.float32),
      mesh=mesh,
      scratch_types=[
          pltpu.VMEM((3, WIN, D), jnp.float32),
          pltpu.VMEM((NWIN, WIN), jnp.int32),
          pltpu.VMEM_SHARED((BP, D), jnp.float32),
          pltpu.SemaphoreType.DMA,
          pltpu.SemaphoreType.DMA,
          pltpu.SemaphoreType.DMA,
          pltpu.SemaphoreType.DMA,
          pltpu.SemaphoreType.DMA,
          pltpu.SemaphoreType.DMA,
          pltpu.SemaphoreType.DMA,
      ],
  )
  def k(hyps_hbm, ids_hbm, out_hbm, rows_v, idx_v, acc_sh,
        seml0, seml1, seml2, sems0, sems1, sems2, sem_idx):
    cid = lax.axis_index("c")
    sid = lax.axis_index("s")
    tile = cid * NS + sid
    load_sems = (seml0, seml1, seml2)
    scat_sems = (sems0, sems1, sems2)
    chunk_base = tile * ROWS_PER_TILE

    def start_load(w, b):
      pltpu.async_copy(
          hyps_hbm.at[pl.ds(chunk_base + w * WIN, WIN)], rows_v.at[b],
          load_sems[b])

    def wait_load(w, b):
      pltpu.make_async_copy(
          hyps_hbm.at[pl.ds(chunk_base + w * WIN, WIN)], rows_v.at[b],
          load_sems[b]).wait()

    def start_scatter(w, b):
      pltpu.async_copy(rows_v.at[b], acc_sh.at[idx_v.at[w]], scat_sems[b],
                       add=True)

    def wait_scatter(w, b):
      pltpu.make_async_copy(rows_v.at[b], acc_sh.at[idx_v.at[w]],
                            scat_sems[b]).wait()

    idx_load = pltpu.async_copy(ids_hbm.at[tile], idx_v, sem_idx)
    start_load(0, 0)
    start_load(1, 1)

    def zero_body(t, _):
      i = t // (D // 16)
      j = t % (D // 16)
      rows_v[2, i, pl.ds(j * 16, 16)] = jnp.zeros((16,), jnp.float32)
      return 0
    lax.fori_loop(0, ZR * (D // 16), zero_body, 0)

    tile_out_base = sid * ROWS_PER_TILE_B
    for z in range(NZ):
      pltpu.sync_copy(rows_v.at[2],
                      acc_sh.at[pl.ds(tile_out_base + z * ZR, ZR)])

    idx_load.wait()
    plsc.subcore_barrier()

    def win_body(i, _):
      for j in range(3):
        w = i * 3 + j
        b = j
        wait_load(w, b)
        start_scatter(w, b)

        @pl.when(w >= 1)
        def _():
          wait_scatter(w - 1, (j - 1) % 3)
        start_load(w + 2, (j + 2) % 3)
      return 0
    lax.fori_loop(0, (NWIN - 2) // 3, win_body, 0)

    w0, w1 = NWIN - 2, NWIN - 1
    b0, b1 = w0 % 3, w1 % 3
    wait_load(w0, b0)
    start_scatter(w0, b0)
    wait_load(w1, b1)
    start_scatter(w1, b1)
    wait_scatter(w1 - 2, (b1 - 2) % 3)
    wait_scatter(w0, b0)
    wait_scatter(w1, b1)

    plsc.subcore_barrier()

    pltpu.sync_copy(acc_sh.at[pl.ds(tile_out_base, ROWS_PER_TILE_B)],
                    out_hbm.at[cid, pl.ds(tile_out_base, ROWS_PER_TILE_B)])

  return k(hyps, seg_ids)


def _mlp_body(goals_ref, p_ref, w1_ref, b1_ref, w2_ref, b2_ref,
              w3_ref, b3_ref, out_ref):
  hsum = p_ref[0] + p_ref[1]
  x = jnp.concatenate([goals_ref[...], hsum], axis=1).astype(jnp.bfloat16)
  z1 = lax.dot_general(x, w1_ref[...].astype(jnp.bfloat16),
                       (((1,), (1,)), ((), ())),
                       preferred_element_type=jnp.float32) + b1_ref[...]
  a1 = jnp.where(z1 >= 0, z1, 0.01 * z1).astype(jnp.bfloat16)
  z2 = lax.dot_general(a1, w2_ref[...].astype(jnp.bfloat16),
                       (((1,), (1,)), ((), ())),
                       preferred_element_type=jnp.float32) + b2_ref[...]
  a2 = jnp.where(z2 >= 0, z2, 0.01 * z2)
  z3 = jnp.sum(a2 * w3_ref[...], axis=1, keepdims=True) + b3_ref[0, 0]
  out_ref[...] = jax.nn.sigmoid(z3)


def _mlp_tc(goals, partials, W1, b1, W2, b2, W3, b3):
  R = 1000
  grid = B // R
  full = lambda shape: pl.BlockSpec(shape, lambda i: (0, 0))
  return pl.pallas_call(
      _mlp_body,
      grid=(grid,),
      in_specs=[
          pl.BlockSpec((R, D), lambda i: (i, 0)),
          pl.BlockSpec((NC, R, D), lambda i: (0, i, 0)),
          full(W1.shape),
          full((1, H)),
          full(W2.shape),
          full((1, H)),
          full(W3.shape),
          full((1, 1)),
      ],
      out_specs=pl.BlockSpec((R, 1), lambda i: (i, 0)),
      out_shape=jax.ShapeDtypeStruct((B, 1), jnp.float32),
  )(goals, partials, W1, b1.reshape(1, H), W2, b2.reshape(1, H), W3,
    b3.reshape(1, 1))


def kernel(goals, hyps, segment_ids, W1, b1, W2, b2, W3, b3):
  seg = segment_ids.astype(jnp.int32).reshape(NW, NWIN, WIN)
  partials = _segment_sum_sc(hyps, seg)
  return _mlp_tc(goals, partials, W1, b1, W2, b2, W3, b3)


# speedup vs baseline: 7.2280x; 1.0064x over previous
import functools

import jax
import jax.numpy as jnp
from jax import lax
from jax.experimental import pallas as pl
from jax.experimental.pallas import tpu as pltpu
from jax.experimental.pallas import tpu_sc as plsc

B, D, E = 10000, 128, 320000
H = 256
BP = 10240

NC, NS = 2, 16
NW = NC * NS
ROWS_PER_TILE = E // NW
WIN = 80
NWIN = ROWS_PER_TILE // WIN
ZR = WIN
ROWS_PER_TILE_B = BP // NS
NZ = ROWS_PER_TILE_B // ZR


def _segment_sum_sc(hyps, seg_ids):
  mesh = plsc.VectorSubcoreMesh(
      core_axis_name="c", subcore_axis_name="s", num_cores=NC,
      num_subcores=NS)

  @functools.partial(
      pl.kernel,
      out_type=jax.ShapeDtypeStruct((NC, BP, D), jnp.float32),
      mesh=mesh,
      scratch_types=[
          pltpu.VMEM((3, WIN, D), jnp.float32),
          pltpu.VMEM((NWIN, WIN), jnp.int32),
          pltpu.VMEM_SHARED((BP, D), jnp.float32),
          pltpu.SemaphoreType.DMA,
          pltpu.SemaphoreType.DMA,
          pltpu.SemaphoreType.DMA,
          pltpu.SemaphoreType.DMA,
          pltpu.SemaphoreType.DMA,
          pltpu.SemaphoreType.DMA,
          pltpu.SemaphoreType.DMA,
      ],
  )
  def k(hyps_hbm, ids_hbm, out_hbm, rows_v, idx_v, acc_sh,
        seml0, seml1, seml2, sems0, sems1, sems2, sem_idx):
    cid = lax.axis_index("c")
    sid = lax.axis_index("s")
    tile = cid * NS + sid
    load_sems = (seml0, seml1, seml2)
    scat_sems = (sems0, sems1, sems2)
    chunk_base = tile * ROWS_PER_TILE

    def start_load(w, b):
      pltpu.async_copy(
          hyps_hbm.at[pl.ds(chunk_base + w * WIN, WIN)], rows_v.at[b],
          load_sems[b])

    def wait_load(w, b):
      pltpu.make_async_copy(
          hyps_hbm.at[pl.ds(chunk_base + w * WIN, WIN)], rows_v.at[b],
          load_sems[b]).wait()

    def start_scatter(w, b):
      pltpu.async_copy(rows_v.at[b], acc_sh.at[idx_v.at[w]], scat_sems[b],
                       add=True)

    def wait_scatter(w, b):
      pltpu.make_async_copy(rows_v.at[b], acc_sh.at[idx_v.at[w]],
                            scat_sems[b]).wait()

    idx_load = pltpu.async_copy(ids_hbm.at[tile], idx_v, sem_idx)
    start_load(0, 0)
    start_load(1, 1)

    zeros16 = jnp.zeros((16,), jnp.float32)
    def zero_body(i, _):
      for j in range(D // 16):
        rows_v[2, i, pl.ds(j * 16, 16)] = zeros16
      return 0
    lax.fori_loop(0, ZR, zero_body, 0)

    tile_out_base = sid * ROWS_PER_TILE_B
    for z in range(NZ):
      pltpu.async_copy(rows_v.at[2],
                       acc_sh.at[pl.ds(tile_out_base + z * ZR, ZR)],
                       sems2)
    for z in range(NZ):
      pltpu.make_async_copy(
          rows_v.at[2], acc_sh.at[pl.ds(tile_out_base + z * ZR, ZR)],
          sems2).wait()

    idx_load.wait()
    plsc.subcore_barrier()

    def win_body(i, _):
      for j in range(3):
        w = i * 3 + j
        b = j
        wait_load(w, b)
        start_scatter(w, b)

        @pl.when(w >= 1)
        def _():
          wait_scatter(w - 1, (j - 1) % 3)
        start_load(w + 2, (j + 2) % 3)
      return 0
    lax.fori_loop(0, (NWIN - 2) // 3, win_body, 0)

    w0, w1 = NWIN - 2, NWIN - 1
    b0, b1 = w0 % 3, w1 % 3
    wait_load(w0, b0)
    start_scatter(w0, b0)
    wait_load(w1, b1)
    start_scatter(w1, b1)
    wait_scatter(w1 - 2, (b1 - 2) % 3)
    wait_scatter(w0, b0)
    wait_scatter(w1, b1)

    plsc.subcore_barrier()

    pltpu.sync_copy(acc_sh.at[pl.ds(tile_out_base, ROWS_PER_TILE_B)],
                    out_hbm.at[cid, pl.ds(tile_out_base, ROWS_PER_TILE_B)])

  return k(hyps, seg_ids)


def _mlp_body(goals_ref, p_ref, w1_ref, b1_ref, w2_ref, b2_ref,
              w3_ref, b3_ref, out_ref):
  hsum = p_ref[0] + p_ref[1]
  x = jnp.concatenate([goals_ref[...], hsum], axis=1).astype(jnp.bfloat16)
  z1 = lax.dot_general(x, w1_ref[...].astype(jnp.bfloat16),
                       (((1,), (1,)), ((), ())),
                       preferred_element_type=jnp.float32) + b1_ref[...]
  a1 = jnp.where(z1 >= 0, z1, 0.01 * z1).astype(jnp.bfloat16)
  z2 = lax.dot_general(a1, w2_ref[...].astype(jnp.bfloat16),
                       (((1,), (1,)), ((), ())),
                       preferred_element_type=jnp.float32) + b2_ref[...]
  a2 = jnp.where(z2 >= 0, z2, 0.01 * z2)
  z3 = jnp.sum(a2 * w3_ref[...], axis=1, keepdims=True) + b3_ref[0, 0]
  out_ref[...] = jax.nn.sigmoid(z3)


def _mlp_tc(goals, partials, W1, b1, W2, b2, W3, b3):
  R = 1000
  grid = B // R
  full = lambda shape: pl.BlockSpec(shape, lambda i: (0, 0))
  return pl.pallas_call(
      _mlp_body,
      grid=(grid,),
      in_specs=[
          pl.BlockSpec((R, D), lambda i: (i, 0)),
          pl.BlockSpec((NC, R, D), lambda i: (0, i, 0)),
          full(W1.shape),
          full((1, H)),
          full(W2.shape),
          full((1, H)),
          full(W3.shape),
          full((1, 1)),
      ],
      out_specs=pl.BlockSpec((R, 1), lambda i: (i, 0)),
      out_shape=jax.ShapeDtypeStruct((B, 1), jnp.float32),
  )(goals, partials, W1, b1.reshape(1, H), W2, b2.reshape(1, H), W3,
    b3.reshape(1, 1))


def kernel(goals, hyps, segment_ids, W1, b1, W2, b2, W3, b3):
  seg = segment_ids.astype(jnp.int32).reshape(NW, NWIN, WIN)
  partials = _segment_sum_sc(hyps, seg)
  return _mlp_tc(goals, partials, W1, b1, W2, b2, W3, b3)


# speedup vs baseline: 7.6106x; 1.0529x over previous
import functools

import jax
import jax.numpy as jnp
from jax import lax
from jax.experimental import pallas as pl
from jax.experimental.pallas import tpu as pltpu
from jax.experimental.pallas import tpu_sc as plsc

B, D, E = 10000, 128, 320000
H = 256
BP = 10240

NC, NS = 2, 16
NW = NC * NS
ROWS_PER_TILE = E // NW
WIN = 80
NWIN = ROWS_PER_TILE // WIN
ZR = WIN
ROWS_PER_TILE_B = BP // NS
NZ = ROWS_PER_TILE_B // ZR


def _segment_sum_sc(hyps, seg_ids):
  mesh = plsc.VectorSubcoreMesh(
      core_axis_name="c", subcore_axis_name="s", num_cores=NC,
      num_subcores=NS)

  @functools.partial(
      pl.kernel,
      out_type=jax.ShapeDtypeStruct((NC, BP, D), jnp.float32),
      mesh=mesh,
      scratch_types=[
          pltpu.VMEM((4, WIN, D), jnp.float32),
          pltpu.VMEM((4, 1, WIN), jnp.int32),
          pltpu.VMEM_SHARED((BP, D), jnp.float32),
          pltpu.SemaphoreType.DMA,
          pltpu.SemaphoreType.DMA,
          pltpu.SemaphoreType.DMA,
          pltpu.SemaphoreType.DMA,
          pltpu.SemaphoreType.DMA,
          pltpu.SemaphoreType.DMA,
          pltpu.SemaphoreType.DMA,
          pltpu.SemaphoreType.DMA,
      ],
  )
  def k(hyps_hbm, ids_hbm, out_hbm, rows_v, idx_v, acc_sh,
        seml0, seml1, seml2, seml3, sems0, sems1, sems2, sems3):
    cid = lax.axis_index("c")
    sid = lax.axis_index("s")
    tile = cid * NS + sid
    load_sems = (seml0, seml1, seml2, seml3)
    scat_sems = (sems0, sems1, sems2, sems3)
    chunk_base = tile * ROWS_PER_TILE

    def start_load(w, b):
      pltpu.async_copy(
          hyps_hbm.at[pl.ds(chunk_base + w * WIN, WIN)], rows_v.at[b],
          load_sems[b])
      pltpu.async_copy(ids_hbm.at[tile, w], idx_v.at[b], load_sems[b])

    def wait_load(w, b):
      pltpu.make_async_copy(
          hyps_hbm.at[pl.ds(chunk_base + w * WIN, WIN)], rows_v.at[b],
          load_sems[b]).wait()
      pltpu.make_async_copy(ids_hbm.at[tile, w], idx_v.at[b],
                            load_sems[b]).wait()

    def start_scatter(b):
      pltpu.async_copy(rows_v.at[b], acc_sh.at[idx_v.at[b, 0]],
                       scat_sems[b], add=True)

    def wait_scatter(b):
      pltpu.make_async_copy(rows_v.at[b], acc_sh.at[idx_v.at[b, 0]],
                            scat_sems[b]).wait()

    start_load(0, 0)
    start_load(1, 1)
    start_load(2, 2)

    zeros16 = jnp.zeros((16,), jnp.float32)
    def zero_body(i, _):
      for j in range(D // 16):
        rows_v[3, i, pl.ds(j * 16, 16)] = zeros16
      return 0
    lax.fori_loop(0, ZR, zero_body, 0)

    tile_out_base = sid * ROWS_PER_TILE_B
    for z in range(NZ):
      pltpu.async_copy(rows_v.at[3],
                       acc_sh.at[pl.ds(tile_out_base + z * ZR, ZR)],
                       sems3)
    for z in range(NZ):
      pltpu.make_async_copy(
          rows_v.at[3], acc_sh.at[pl.ds(tile_out_base + z * ZR, ZR)],
          sems3).wait()

    plsc.subcore_barrier()

    def win_body(i, _):
      for j in range(4):
        w = i * 4 + j
        b = j
        wait_load(w, b)
        start_scatter(b)

        @pl.when(w >= 1)
        def _():
          wait_scatter((j - 1) % 4)

        @pl.when(w + 3 < NWIN)
        def _():
          start_load(w + 3, (j + 3) % 4)
      return 0
    lax.fori_loop(0, NWIN // 4, win_body, 0)

    wl, bl = NWIN - 1, (NWIN - 1) % 4
    wait_load(wl, bl)
    start_scatter(bl)
    wait_scatter((bl - 1) % 4)
    wait_scatter(bl)

    plsc.subcore_barrier()

    pltpu.sync_copy(acc_sh.at[pl.ds(tile_out_base, ROWS_PER_TILE_B)],
                    out_hbm.at[cid, pl.ds(tile_out_base, ROWS_PER_TILE_B)])

  return k(hyps, seg_ids)


def _mlp_body(goals_ref, p_ref, w1_ref, b1_ref, w2_ref, b2_ref,
              w3_ref, b3_ref, out_ref):
  hsum = p_ref[0] + p_ref[1]
  x = jnp.concatenate([goals_ref[...], hsum], axis=1).astype(jnp.bfloat16)
  z1 = lax.dot_general(x, w1_ref[...].astype(jnp.bfloat16),
                       (((1,), (1,)), ((), ())),
                       preferred_element_type=jnp.float32) + b1_ref[...]
  a1 = jnp.where(z1 >= 0, z1, 0.01 * z1).astype(jnp.bfloat16)
  z2 = lax.dot_general(a1, w2_ref[...].astype(jnp.bfloat16),
                       (((1,), (1,)), ((), ())),
                       preferred_element_type=jnp.float32) + b2_ref[...]
  a2 = jnp.where(z2 >= 0, z2, 0.01 * z2)
  z3 = jnp.sum(a2 * w3_ref[...], axis=1, keepdims=True) + b3_ref[0, 0]
  out_ref[...] = jax.nn.sigmoid(z3)


def _mlp_tc(goals, partials, W1, b1, W2, b2, W3, b3):
  R = 1000
  grid = B // R
  full = lambda shape: pl.BlockSpec(shape, lambda i: (0, 0))
  return pl.pallas_call(
      _mlp_body,
      grid=(grid,),
      in_specs=[
          pl.BlockSpec((R, D), lambda i: (i, 0)),
          pl.BlockSpec((NC, R, D), lambda i: (0, i, 0)),
          full(W1.shape),
          full((1, H)),
          full(W2.shape),
          full((1, H)),
          full(W3.shape),
          full((1, 1)),
      ],
      out_specs=pl.BlockSpec((R, 1), lambda i: (i, 0)),
      out_shape=jax.ShapeDtypeStruct((B, 1), jnp.float32),
  )(goals, partials, W1, b1.reshape(1, H), W2, b2.reshape(1, H), W3,
    b3.reshape(1, 1))


def kernel(goals, hyps, segment_ids, W1, b1, W2, b2, W3, b3):
  seg = segment_ids.astype(jnp.int32).reshape(NW, NWIN, 1, WIN)
  partials = _segment_sum_sc(hyps, seg)
  return _mlp_tc(goals, partials, W1, b1, W2, b2, W3, b3)


# speedup vs baseline: 8.2063x; 1.0783x over previous
import functools

import jax
import jax.numpy as jnp
from jax import lax
from jax.experimental import pallas as pl
from jax.experimental.pallas import tpu as pltpu
from jax.experimental.pallas import tpu_sc as plsc

B, D, E = 10000, 128, 320000
H = 256
BP = 10240

NC, NS = 2, 16
NW = NC * NS
ROWS_PER_TILE = E // NW
WIN = 80
NWIN = ROWS_PER_TILE // WIN
ZR = WIN
ROWS_PER_TILE_B = BP // NS
NZ = ROWS_PER_TILE_B // ZR


def _segment_sum_sc(hyps, seg_ids):
  mesh = plsc.VectorSubcoreMesh(
      core_axis_name="c", subcore_axis_name="s", num_cores=NC,
      num_subcores=NS)

  @functools.partial(
      pl.kernel,
      out_type=jax.ShapeDtypeStruct((NC, BP, D), jnp.float32),
      mesh=mesh,
      scratch_types=[
          pltpu.VMEM((4, WIN, D), jnp.float32),
          pltpu.VMEM((4, 1, WIN), jnp.int32),
          pltpu.VMEM_SHARED((BP, D), jnp.float32),
          pltpu.SemaphoreType.DMA,
          pltpu.SemaphoreType.DMA,
          pltpu.SemaphoreType.DMA,
          pltpu.SemaphoreType.DMA,
          pltpu.SemaphoreType.DMA,
          pltpu.SemaphoreType.DMA,
          pltpu.SemaphoreType.DMA,
          pltpu.SemaphoreType.DMA,
      ],
  )
  def k(hyps_hbm, ids_hbm, out_hbm, rows_v, idx_v, acc_sh,
        seml0, seml1, seml2, seml3, sems0, sems1, sems2, sems3):
    cid = lax.axis_index("c")
    sid = lax.axis_index("s")
    tile = cid * NS + sid
    load_sems = (seml0, seml1, seml2, seml3)
    scat_sems = (sems0, sems1, sems2, sems3)
    chunk_base = tile * ROWS_PER_TILE

    def start_load(w, b):
      pltpu.async_copy(
          hyps_hbm.at[pl.ds(chunk_base + w * WIN, WIN)], rows_v.at[b],
          load_sems[b])
      pltpu.async_copy(ids_hbm.at[pl.ds(chunk_base + w * WIN, WIN)],
                       idx_v.at[b, 0], load_sems[b])

    def wait_load(w, b):
      pltpu.make_async_copy(
          hyps_hbm.at[pl.ds(chunk_base + w * WIN, WIN)], rows_v.at[b],
          load_sems[b]).wait()
      pltpu.make_async_copy(ids_hbm.at[pl.ds(chunk_base + w * WIN, WIN)],
                            idx_v.at[b, 0], load_sems[b]).wait()

    def start_scatter(b):
      pltpu.async_copy(rows_v.at[b], acc_sh.at[idx_v.at[b, 0]],
                       scat_sems[b], add=True)

    def wait_scatter(b):
      pltpu.make_async_copy(rows_v.at[b], acc_sh.at[idx_v.at[b, 0]],
                            scat_sems[b]).wait()

    start_load(0, 0)
    start_load(1, 1)
    start_load(2, 2)

    zeros16 = jnp.zeros((16,), jnp.float32)
    def zero_body(i, _):
      for j in range(D // 16):
        rows_v[3, i, pl.ds(j * 16, 16)] = zeros16
      return 0
    lax.fori_loop(0, ZR, zero_body, 0)

    tile_out_base = sid * ROWS_PER_TILE_B
    for z in range(NZ):
      pltpu.async_copy(rows_v.at[3],
                       acc_sh.at[pl.ds(tile_out_base + z * ZR, ZR)],
                       sems3)
    for z in range(NZ):
      pltpu.make_async_copy(
          rows_v.at[3], acc_sh.at[pl.ds(tile_out_base + z * ZR, ZR)],
          sems3).wait()

    plsc.subcore_barrier()

    def win_body(i, _):
      for j in range(4):
        w = i * 4 + j
        b = j
        wait_load(w, b)
        start_scatter(b)

        @pl.when(w >= 1)
        def _():
          wait_scatter((j - 1) % 4)

        @pl.when(w + 3 < NWIN)
        def _():
          start_load(w + 3, (j + 3) % 4)
      return 0
    lax.fori_loop(0, NWIN // 4, win_body, 0)

    wl, bl = NWIN - 1, (NWIN - 1) % 4
    wait_load(wl, bl)
    start_scatter(bl)
    wait_scatter((bl - 1) % 4)
    wait_scatter(bl)

    plsc.subcore_barrier()

    pltpu.sync_copy(acc_sh.at[pl.ds(tile_out_base, ROWS_PER_TILE_B)],
                    out_hbm.at[cid, pl.ds(tile_out_base, ROWS_PER_TILE_B)])

  return k(hyps, seg_ids)


def _mlp_body(goals_ref, p_ref, w1_ref, b1_ref, w2_ref, b2_ref,
              w3_ref, b3_ref, out_ref):
  hsum = p_ref[0] + p_ref[1]
  x = jnp.concatenate([goals_ref[...], hsum], axis=1).astype(jnp.bfloat16)
  z1 = lax.dot_general(x, w1_ref[...].astype(jnp.bfloat16),
                       (((1,), (1,)), ((), ())),
                       preferred_element_type=jnp.float32) + b1_ref[...]
  a1 = jnp.where(z1 >= 0, z1, 0.01 * z1).astype(jnp.bfloat16)
  z2 = lax.dot_general(a1, w2_ref[...].astype(jnp.bfloat16),
                       (((1,), (1,)), ((), ())),
                       preferred_element_type=jnp.float32) + b2_ref[...]
  a2 = jnp.where(z2 >= 0, z2, 0.01 * z2)
  z3 = jnp.sum(a2 * w3_ref[...], axis=1) + b3_ref[0, 0]
  out_ref[...] = jax.nn.sigmoid(z3).reshape(out_ref.shape)


def _mlp_tc(goals, partials, W1, b1, W2, b2, W3, b3):
  R = 1024
  grid = BP // R
  full = lambda shape: pl.BlockSpec(shape, lambda i: (0, 0))
  out2 = pl.pallas_call(
      _mlp_body,
      grid=(grid,),
      in_specs=[
          pl.BlockSpec((R, D), lambda i: (i, 0)),
          pl.BlockSpec((NC, R, D), lambda i: (0, i, 0)),
          full(W1.shape),
          full((1, H)),
          full(W2.shape),
          full((1, H)),
          full(W3.shape),
          full((1, 1)),
      ],
      out_specs=pl.BlockSpec((R // 128, 128), lambda i: (i, 0)),
      out_shape=jax.ShapeDtypeStruct((BP // 128, 128), jnp.float32),
  )(goals, partials, W1, b1.reshape(1, H), W2, b2.reshape(1, H), W3,
    b3.reshape(1, 1))
  return out2.reshape(BP)[:B].reshape(B, 1)


def kernel(goals, hyps, segment_ids, W1, b1, W2, b2, W3, b3):
  seg = segment_ids.astype(jnp.int32)
  partials = _segment_sum_sc(hyps, seg)
  return _mlp_tc(goals, partials, W1, b1, W2, b2, W3, b3)


# speedup vs baseline: 8.3583x; 1.0185x over previous
import functools

import jax
import jax.numpy as jnp
from jax import lax
from jax.experimental import pallas as pl
from jax.experimental.pallas import tpu as pltpu
from jax.experimental.pallas import tpu_sc as plsc

B, D, E = 10000, 128, 320000
H = 256
BP = 10240

NC, NS = 2, 16
NW = NC * NS
ROWS_PER_TILE = E // NW
WIN = 80
NWIN = ROWS_PER_TILE // WIN
ZR = WIN
ROWS_PER_TILE_B = BP // NS
NZ = ROWS_PER_TILE_B // ZR


def _segment_sum_sc(hyps, seg_ids):
  mesh = plsc.VectorSubcoreMesh(
      core_axis_name="c", subcore_axis_name="s", num_cores=NC,
      num_subcores=NS)

  @functools.partial(
      pl.kernel,
      out_type=jax.ShapeDtypeStruct((NC, BP, D), jnp.float32),
      mesh=mesh,
      scratch_types=[
          pltpu.VMEM((4, WIN, D), jnp.float32),
          pltpu.VMEM((4, 1, WIN), jnp.int32),
          pltpu.VMEM_SHARED((BP, D), jnp.float32),
          pltpu.SemaphoreType.DMA,
          pltpu.SemaphoreType.DMA,
          pltpu.SemaphoreType.DMA,
          pltpu.SemaphoreType.DMA,
          pltpu.SemaphoreType.DMA,
          pltpu.SemaphoreType.DMA,
          pltpu.SemaphoreType.DMA,
          pltpu.SemaphoreType.DMA,
      ],
  )
  def k(hyps_hbm, ids_hbm, out_hbm, rows_v, idx_v, acc_sh,
        seml0, seml1, seml2, seml3, sems0, sems1, sems2, sems3):
    cid = lax.axis_index("c")
    sid = lax.axis_index("s")
    tile = cid * NS + sid
    load_sems = (seml0, seml1, seml2, seml3)
    scat_sems = (sems0, sems1, sems2, sems3)
    chunk_base = tile * ROWS_PER_TILE

    def start_load(w, b):
      pltpu.async_copy(
          hyps_hbm.at[pl.ds(chunk_base + w * WIN, WIN)], rows_v.at[b],
          load_sems[b])
      pltpu.async_copy(ids_hbm.at[pl.ds(chunk_base + w * WIN, WIN)],
                       idx_v.at[b, 0], load_sems[b])

    def wait_load(w, b):
      pltpu.make_async_copy(
          hyps_hbm.at[pl.ds(chunk_base + w * WIN, WIN)], rows_v.at[b],
          load_sems[b]).wait()
      pltpu.make_async_copy(ids_hbm.at[pl.ds(chunk_base + w * WIN, WIN)],
                            idx_v.at[b, 0], load_sems[b]).wait()

    def start_scatter(b):
      pltpu.async_copy(rows_v.at[b], acc_sh.at[idx_v.at[b, 0]],
                       scat_sems[b], add=True)

    def wait_scatter(b):
      pltpu.make_async_copy(rows_v.at[b], acc_sh.at[idx_v.at[b, 0]],
                            scat_sems[b]).wait()

    start_load(0, 0)
    start_load(1, 1)
    start_load(2, 2)

    zeros16 = jnp.zeros((16,), jnp.float32)
    def zero_body(i, _):
      for j in range(D // 16):
        rows_v[3, i, pl.ds(j * 16, 16)] = zeros16
      return 0
    lax.fori_loop(0, ZR, zero_body, 0)

    tile_out_base = sid * ROWS_PER_TILE_B
    for z in range(NZ):
      pltpu.async_copy(rows_v.at[3],
                       acc_sh.at[pl.ds(tile_out_base + z * ZR, ZR)],
                       sems3)
    for z in range(NZ):
      pltpu.make_async_copy(
          rows_v.at[3], acc_sh.at[pl.ds(tile_out_base + z * ZR, ZR)],
          sems3).wait()

    plsc.subcore_barrier()

    def win_body(i, _):
      for j in range(4):
        w = i * 4 + j
        b = j
        wait_load(w, b)
        start_scatter(b)

        @pl.when(w >= 1)
        def _():
          wait_scatter((j - 1) % 4)

        @pl.when(w + 3 < NWIN)
        def _():
          start_load(w + 3, (j + 3) % 4)
      return 0
    lax.fori_loop(0, NWIN // 4, win_body, 0)

    wl, bl = NWIN - 1, (NWIN - 1) % 4
    wait_load(wl, bl)
    start_scatter(bl)
    wait_scatter((bl - 1) % 4)
    wait_scatter(bl)

    plsc.subcore_barrier()

    pltpu.sync_copy(acc_sh.at[pl.ds(tile_out_base, ROWS_PER_TILE_B)],
                    out_hbm.at[cid, pl.ds(tile_out_base, ROWS_PER_TILE_B)])

  return k(hyps, seg_ids)


def _mlp_body(goals_ref, p_ref, w1_ref, b1_ref, w2_ref, b2_ref,
              w3_ref, b3_ref, out_ref):
  hsum = p_ref[0] + p_ref[1]
  x = jnp.concatenate([goals_ref[...], hsum], axis=1).astype(jnp.bfloat16)
  z1 = lax.dot_general(x, w1_ref[...].astype(jnp.bfloat16),
                       (((1,), (1,)), ((), ())),
                       preferred_element_type=jnp.float32) + b1_ref[...]
  a1 = jnp.where(z1 >= 0, z1, 0.01 * z1).astype(jnp.bfloat16)
  z2 = lax.dot_general(a1, w2_ref[...].astype(jnp.bfloat16),
                       (((1,), (1,)), ((), ())),
                       preferred_element_type=jnp.float32) + b2_ref[...]
  a2 = jnp.where(z2 >= 0, z2, 0.01 * z2)
  z3 = jnp.sum(a2 * w3_ref[...], axis=1) + b3_ref[0, 0]
  out_ref[...] = jax.nn.sigmoid(z3).reshape(out_ref.shape)


def _mlp_tc(goals, partials, W1, b1, W2, b2, W3, b3):
  R = 2048
  grid = BP // R
  full = lambda shape: pl.BlockSpec(shape, lambda i: (0, 0))
  out2 = pl.pallas_call(
      _mlp_body,
      grid=(grid,),
      in_specs=[
          pl.BlockSpec((R, D), lambda i: (i, 0)),
          pl.BlockSpec((NC, R, D), lambda i: (0, i, 0)),
          full(W1.shape),
          full((1, H)),
          full(W2.shape),
          full((1, H)),
          full(W3.shape),
          full((1, 1)),
      ],
      out_specs=pl.BlockSpec((R // 128, 128), lambda i: (i, 0)),
      out_shape=jax.ShapeDtypeStruct((BP // 128, 128), jnp.float32),
  )(goals, partials, W1, b1.reshape(1, H), W2, b2.reshape(1, H), W3,
    b3.reshape(1, 1))
  return out2.reshape(BP)[:B].reshape(B, 1)


def kernel(goals, hyps, segment_ids, W1, b1, W2, b2, W3, b3):
  seg = segment_ids.astype(jnp.int32)
  partials = _segment_sum_sc(hyps, seg)
  return _mlp_tc(goals, partials, W1, b1, W2, b2, W3, b3)


# speedup vs baseline: 8.6750x; 1.0379x over previous
import functools

import jax
import jax.numpy as jnp
from jax import lax
from jax.experimental import pallas as pl
from jax.experimental.pallas import tpu as pltpu
from jax.experimental.pallas import tpu_sc as plsc

B, D, E = 10000, 128, 320000
H = 256
BP = 10240

NC, NS = 2, 16
NW = NC * NS
ROWS_PER_TILE = E // NW
WIN = 40
NWIN = ROWS_PER_TILE // WIN
RING = 8
ZR = WIN
ROWS_PER_TILE_B = BP // NS
NZ = ROWS_PER_TILE_B // ZR


def _segment_sum_sc(hyps, seg_ids):
  mesh = plsc.VectorSubcoreMesh(
      core_axis_name="c", subcore_axis_name="s", num_cores=NC,
      num_subcores=NS)

  @functools.partial(
      pl.kernel,
      out_type=jax.ShapeDtypeStruct((NC, BP, D), jnp.float32),
      mesh=mesh,
      scratch_types=[
          pltpu.VMEM((RING, WIN, D), jnp.float32),
          pltpu.VMEM((RING, 1, WIN), jnp.int32),
          pltpu.VMEM_SHARED((BP, D), jnp.float32),
      ] + [pltpu.SemaphoreType.DMA] * (2 * RING),
  )
  def k(hyps_hbm, ids_hbm, out_hbm, rows_v, idx_v, acc_sh, *sems):
    cid = lax.axis_index("c")
    sid = lax.axis_index("s")
    tile = cid * NS + sid
    load_sems = sems[:RING]
    scat_sems = sems[RING:]
    chunk_base = tile * ROWS_PER_TILE

    def start_load(w, b):
      pltpu.async_copy(
          hyps_hbm.at[pl.ds(chunk_base + w * WIN, WIN)], rows_v.at[b],
          load_sems[b])
      pltpu.async_copy(ids_hbm.at[pl.ds(chunk_base + w * WIN, WIN)],
                       idx_v.at[b, 0], load_sems[b])

    def wait_load(w, b):
      pltpu.make_async_copy(
          hyps_hbm.at[pl.ds(chunk_base + w * WIN, WIN)], rows_v.at[b],
          load_sems[b]).wait()
      pltpu.make_async_copy(ids_hbm.at[pl.ds(chunk_base + w * WIN, WIN)],
                            idx_v.at[b, 0], load_sems[b]).wait()

    def start_scatter(b):
      pltpu.async_copy(rows_v.at[b], acc_sh.at[idx_v.at[b, 0]],
                       scat_sems[b], add=True)

    def wait_scatter(b):
      pltpu.make_async_copy(rows_v.at[b], acc_sh.at[idx_v.at[b, 0]],
                            scat_sems[b]).wait()

    for w in range(RING - 1):
      start_load(w, w)

    zslot = RING - 1
    zeros16 = jnp.zeros((16,), jnp.float32)
    def zero_body(i, _):
      for j in range(D // 16):
        rows_v[zslot, i, pl.ds(j * 16, 16)] = zeros16
      return 0
    lax.fori_loop(0, ZR, zero_body, 0)

    tile_out_base = sid * ROWS_PER_TILE_B
    for z in range(NZ):
      pltpu.async_copy(rows_v.at[zslot],
                       acc_sh.at[pl.ds(tile_out_base + z * ZR, ZR)],
                       scat_sems[zslot])
    for z in range(NZ):
      pltpu.make_async_copy(
          rows_v.at[zslot], acc_sh.at[pl.ds(tile_out_base + z * ZR, ZR)],
          scat_sems[zslot]).wait()

    plsc.subcore_barrier()

    def win_body(i, _):
      for j in range(RING):
        w = i * RING + j
        b = j
        wait_load(w, b)
        start_scatter(b)

        @pl.when(w >= 1)
        def _():
          wait_scatter((j - 1) % RING)

        @pl.when(w + RING - 1 < NWIN)
        def _():
          start_load(w + RING - 1, (j + RING - 1) % RING)
      return 0
    lax.fori_loop(0, NWIN // RING, win_body, 0)

    for w in range((NWIN // RING) * RING, NWIN):
      wait_load(w, w % RING)
      start_scatter(w % RING)
      wait_scatter((w - 1) % RING)
    wait_scatter((NWIN - 1) % RING)

    plsc.subcore_barrier()

    pltpu.sync_copy(acc_sh.at[pl.ds(tile_out_base, ROWS_PER_TILE_B)],
                    out_hbm.at[cid, pl.ds(tile_out_base, ROWS_PER_TILE_B)])

  return k(hyps, seg_ids)


def _mlp_body(goals_ref, p_ref, w1_ref, b1_ref, w2_ref, b2_ref,
              w3_ref, b3_ref, out_ref):
  hsum = p_ref[0] + p_ref[1]
  x = jnp.concatenate([goals_ref[...], hsum], axis=1).astype(jnp.bfloat16)
  z1 = lax.dot_general(x, w1_ref[...].astype(jnp.bfloat16),
                       (((1,), (1,)), ((), ())),
                       preferred_element_type=jnp.float32) + b1_ref[...]
  a1 = jnp.where(z1 >= 0, z1, 0.01 * z1).astype(jnp.bfloat16)
  z2 = lax.dot_general(a1, w2_ref[...].astype(jnp.bfloat16),
                       (((1,), (1,)), ((), ())),
                       preferred_element_type=jnp.float32) + b2_ref[...]
  a2 = jnp.where(z2 >= 0, z2, 0.01 * z2)
  z3 = jnp.sum(a2 * w3_ref[...], axis=1) + b3_ref[0, 0]
  out_ref[...] = jax.nn.sigmoid(z3).reshape(out_ref.shape)


def _mlp_tc(goals, partials, W1, b1, W2, b2, W3, b3):
  R = 2048
  grid = BP // R
  full = lambda shape: pl.BlockSpec(shape, lambda i: (0, 0))
  out2 = pl.pallas_call(
      _mlp_body,
      grid=(grid,),
      in_specs=[
          pl.BlockSpec((R, D), lambda i: (i, 0)),
          pl.BlockSpec((NC, R, D), lambda i: (0, i, 0)),
          full(W1.shape),
          full((1, H)),
          full(W2.shape),
          full((1, H)),
          full(W3.shape),
          full((1, 1)),
      ],
      out_specs=pl.BlockSpec((R // 128, 128), lambda i: (i, 0)),
      out_shape=jax.ShapeDtypeStruct((BP // 128, 128), jnp.float32),
  )(goals, partials, W1, b1.reshape(1, H), W2, b2.reshape(1, H), W3,
    b3.reshape(1, 1))
  return out2.reshape(BP)[:B].reshape(B, 1)


def kernel(goals, hyps, segment_ids, W1, b1, W2, b2, W3, b3):
  seg = segment_ids.astype(jnp.int32)
  partials = _segment_sum_sc(hyps, seg)
  return _mlp_tc(goals, partials, W1, b1, W2, b2, W3, b3)
